# baseline (speedup 1.0000x reference)
import jax
import jax.numpy as jnp
from jax import lax
from jax.experimental import pallas as pl
from jax.experimental.pallas import tpu as pltpu
from jax.experimental.pallas import tpu_sc as plsc

DP = 2
MAX_SAMPLES = 16384
N_DIM = 1024
NW = 32
CH = 128
TOTAL_ROWS = DP * MAX_SAMPLES
ROWS_PER_W = TOTAL_ROWS // NW
BPW = ROWS_PER_W // CH
NBLK = TOTAL_ROWS // CH
ACTS_FLAG = 1 << 20


def _build_table(index, chunk):
    b = jnp.arange(NBLK, dtype=jnp.int32)
    r0 = b * CH
    d = r0 // MAX_SAMPLES
    rdp = r0 % MAX_SAMPLES
    off = (rdp - index) % MAX_SAMPLES
    in_acts = off < chunk
    src_acts = jnp.minimum(d * chunk + off, DP * chunk - CH) + ACTS_FLAG
    src = jnp.where(in_acts, src_acts, r0).astype(jnp.int32)
    return jnp.broadcast_to(src[:, None], (NBLK, 16))


def _copy_body(acts_hbm, cache_hbm, tbl_hbm, out_hbm, tbl_v, sem):
    wid = lax.axis_index("c") * 16 + lax.axis_index("s")
    base = pl.multiple_of(wid * ROWS_PER_W, CH)
    pltpu.sync_copy(tbl_hbm.at[pl.ds(pl.multiple_of(wid * BPW, BPW), BPW)],
                    tbl_v)

    for i in range(BPW):
        s = tbl_v[i][0]
        is_acts = s >= ACTS_FLAG
        dst = out_hbm.at[pl.ds(pl.multiple_of(base + i * CH, CH), CH)]

        @pl.when(is_acts)
        def _():
            pltpu.async_copy(
                acts_hbm.at[pl.ds(pl.multiple_of(s - ACTS_FLAG, 8), CH)],
                dst, sem)

        @pl.when(jnp.logical_not(is_acts))
        def _():
            pltpu.async_copy(cache_hbm.at[pl.ds(pl.multiple_of(s, 8), CH)],
                             dst, sem)

    for i in range(BPW):
        pltpu.make_async_copy(cache_hbm.at[pl.ds(0, CH)],
                              out_hbm.at[pl.ds(base, CH)], sem).wait()


def kernel(activations, cache, n_valid, index):
    dp, max_samples, d = cache.shape
    acts = activations.reshape((dp, -1, d))
    chunk = acts.shape[1]
    new_n_valid = jnp.minimum(jnp.asarray(n_valid) + chunk, max_samples)
    new_index = (jnp.asarray(index) + chunk) % max_samples

    acts_flat = activations.astype(cache.dtype)
    cache_flat = cache.reshape((dp * max_samples, d))
    tbl = _build_table(jnp.asarray(index, dtype=jnp.int32), chunk)

    mesh = plsc.VectorSubcoreMesh(core_axis_name="c", subcore_axis_name="s")
    out_flat = pl.kernel(
        _copy_body,
        mesh=mesh,
        out_type=jax.ShapeDtypeStruct((dp * max_samples, d), cache.dtype),
        scratch_types=[
            pltpu.VMEM((BPW, 16), jnp.int32),
            pltpu.SemaphoreType.DMA,
        ],
    )(acts_flat, cache_flat, tbl)

    new_cache = out_flat.reshape((dp, max_samples, d))
    return (new_cache, new_n_valid, new_index)


# speedup vs baseline: 35.3469x; 35.3469x over previous
import jax
import jax.numpy as jnp
from jax import lax
from jax.experimental import pallas as pl
from jax.experimental.pallas import tpu as pltpu
from jax.experimental.pallas import tpu_sc as plsc

DP = 2
MAX_SAMPLES = 16384
N_DIM = 1024
NW = 32
CH = 16
NB = 6
RPD = 2
TOTAL_ROWS = DP * MAX_SAMPLES
ROWS_PER_W = TOTAL_ROWS // NW
BPW = ROWS_PER_W // CH
NBLK = TOTAL_ROWS // CH
ACTS_FLAG = 1 << 20


def _build_table(index, chunk):
    b = jnp.arange(NBLK, dtype=jnp.int32)
    r0 = b * CH
    d = r0 // MAX_SAMPLES
    rdp = r0 % MAX_SAMPLES
    off = (rdp - index) % MAX_SAMPLES
    in_acts = off < chunk
    src_acts = jnp.minimum(d * chunk + off, DP * chunk - CH) + ACTS_FLAG
    src = jnp.where(in_acts, src_acts, r0).astype(jnp.int32)
    return jnp.broadcast_to(src[:, None], (NBLK, 16))


def _copy_body(acts_hbm, cache_hbm, tbl_hbm, out_hbm, tbl_v, *bufsems):
    wid = lax.axis_index("c") * 16 + lax.axis_index("s")
    base = pl.multiple_of(wid * ROWS_PER_W, CH)
    pltpu.sync_copy(tbl_hbm.at[pl.ds(pl.multiple_of(wid * BPW, 8), BPW)],
                    tbl_v)
    bufs = bufsems[:NB]
    rsems = bufsems[NB:2 * NB]
    wsems = bufsems[2 * NB:]

    def start_read(i, buf, rsem):
        s = tbl_v[i][0]
        is_acts = s >= ACTS_FLAG

        @pl.when(is_acts)
        def _():
            pltpu.async_copy(
                acts_hbm.at[pl.ds(pl.multiple_of(s - ACTS_FLAG, 8), CH)],
                buf, rsem)

        @pl.when(jnp.logical_not(is_acts))
        def _():
            pltpu.async_copy(cache_hbm.at[pl.ds(pl.multiple_of(s, 8), CH)],
                             buf, rsem)

    def wait_read(buf, rsem):
        pltpu.make_async_copy(cache_hbm.at[pl.ds(0, CH)], buf, rsem).wait()

    def wait_write(buf, wsem):
        pltpu.make_async_copy(buf, out_hbm.at[pl.ds(base, CH)], wsem).wait()

    for j in range(RPD):
        start_read(j, bufs[j], rsems[j])

    def body(i, _):
        for p in range(NB):

            @pl.when((i % NB) == p)
            def _():
                q = (p + RPD) % NB

                @pl.when(i + RPD < BPW)
                def _():

                    @pl.when(i + RPD >= NB)
                    def _():
                        wait_write(bufs[q], wsems[q])

                    start_read(i + RPD, bufs[q], rsems[q])

                wait_read(bufs[p], rsems[p])
                pltpu.async_copy(
                    bufs[p],
                    out_hbm.at[pl.ds(pl.multiple_of(base + i * CH, CH), CH)],
                    wsems[p])

        return 0

    lax.fori_loop(0, BPW, body, 0)
    for p in range(NB):
        wait_write(bufs[p], wsems[p])


def kernel(activations, cache, n_valid, index):
    dp, max_samples, d = cache.shape
    acts = activations.reshape((dp, -1, d))
    chunk = acts.shape[1]
    new_n_valid = jnp.minimum(jnp.asarray(n_valid) + chunk, max_samples)
    new_index = (jnp.asarray(index) + chunk) % max_samples

    acts_flat = activations.astype(cache.dtype)
    cache_flat = cache.reshape((dp * max_samples, d))
    tbl = _build_table(jnp.asarray(index, dtype=jnp.int32), chunk)

    mesh = plsc.VectorSubcoreMesh(core_axis_name="c", subcore_axis_name="s")
    out_flat = pl.kernel(
        _copy_body,
        mesh=mesh,
        out_type=jax.ShapeDtypeStruct((dp * max_samples, d), cache.dtype),
        scratch_types=(
            [pltpu.VMEM((BPW, 16), jnp.int32)]
            + [pltpu.VMEM((CH, N_DIM), jnp.float32)] * NB
            + [pltpu.SemaphoreType.DMA] * (2 * NB)
        ),
    )(acts_flat, cache_flat, tbl)

    new_cache = out_flat.reshape((dp, max_samples, d))
    return (new_cache, new_n_valid, new_index)
